# Initial kernel scaffold; baseline (speedup 1.0000x reference)
import jax
import jax.numpy as jnp
from jax.experimental import pallas as pl


def kernel(alpha, mask, basis, missing_basis, alpha_scale, alpha_bias, mask_embedding, value_embedding, semantic_matrix, semantic_proj_w, categorical_value_mask):
    raise NotImplementedError("write your pallas kernel here")


# speedup vs baseline: 5.5549x; 5.5549x over previous
import functools

import jax
import jax.numpy as jnp
from jax import lax
from jax.experimental import pallas as pl
from jax.experimental.pallas import tpu as pltpu
from jax.experimental.pallas import tpu_sc as plsc

B = 4096
N = 100
D = 64
BUCKETS = 256
NW = 32
ROWS = B * N
RPW = ROWS // NW
CHUNK = 400
NCHUNK = RPW // CHUNK
J = 80
TPC = CHUNK // J


def _prep_body(alpha_ref, mask_ref, basis_ref, missing_ref, scale_ref,
               bias_ref, me_ref, ve3_ref, sm_ref, spw_ref, cat_ref,
               a_ref, g_ref, t3_ref):
    sem = lax.dot_general(sm_ref[...], spw_ref[...], (((1,), (1,)), ((), ())),
                          preferred_element_type=jnp.float32)
    me = me_ref[...]
    c1 = sem + me[1:2, :]
    c0 = sem + me[0:1, :] + missing_ref[...]
    catf = cat_ref[...]
    t3_ref[0:N] = (ve3_ref[...] * catf[:, None, None]
                   + c1[:, None, :])
    t3_ref[N:N + 1] = jnp.concatenate(
        [c0, jnp.zeros((BUCKETS - N, D), jnp.float32)], axis=0)[None]

    alpha = alpha_ref[...]
    mask = mask_ref[...]
    mask_f = mask.astype(jnp.float32)
    a_ref[...] = mask_f * (alpha * scale_ref[...][None, :]
                           + bias_ref[...][None, :])
    bucket = jnp.clip(
        jnp.round((jnp.clip(alpha, -1.0, 1.0) + 1.0) * 0.5 * (BUCKETS - 1)),
        0, BUCKETS - 1).astype(jnp.int32)
    n_iota = lax.broadcasted_iota(jnp.int32, alpha.shape, 1)
    g_ref[...] = jnp.where(mask == 1, n_iota * BUCKETS + bucket,
                           N * BUCKETS + n_iota)


def _sc_body(a_hbm, g_hbm, basis_hbm, t_hbm, out_hbm,
             a_v, g_v, basis_v, buf, sem):
    wid = lax.axis_index("s") * 2 + lax.axis_index("c")
    base = wid * RPW
    pltpu.sync_copy(a_hbm.at[wid], a_v)
    pltpu.sync_copy(g_hbm.at[wid], g_v)
    pltpu.sync_copy(basis_hbm, basis_v)

    def chunk_body(c, _):
        for jj in range(TPC):
            t = c * TPC + jj
            pltpu.async_copy(
                t_hbm.at[g_v.at[t]],
                buf.at[pl.ds(jj * J, J)], sem).wait()
        def n_body(n, _):
            bvecs = [basis_v[n, pl.ds(k * 16, 16)] for k in range(4)]
            for j in range(TPC):
                i = n + j * N
                av = plsc.load_gather(
                    a_v, [jnp.full((16,), c * CHUNK + i, jnp.int32)])
                for k in range(4):
                    sl = pl.ds(k * 16, 16)
                    buf[i, sl] = buf[i, sl] + av * bvecs[k]
            return 0
        lax.fori_loop(0, N, n_body, 0)
        pltpu.sync_copy(buf, out_hbm.at[pl.ds(base + c * CHUNK, CHUNK)])
        return 0

    lax.fori_loop(0, NCHUNK, chunk_body, 0)


def kernel(alpha, mask, basis, missing_basis, alpha_scale, alpha_bias,
           mask_embedding, value_embedding, semantic_matrix, semantic_proj_w,
           categorical_value_mask):
    ve3 = value_embedding.reshape(N, BUCKETS, D)
    catf = categorical_value_mask.astype(jnp.float32)
    a, g, t3 = pl.pallas_call(
        _prep_body,
        out_shape=(
            jax.ShapeDtypeStruct((B, N), jnp.float32),
            jax.ShapeDtypeStruct((B, N), jnp.int32),
            jax.ShapeDtypeStruct((N + 1, BUCKETS, D), jnp.float32),
        ),
    )(alpha, mask, basis, missing_basis, alpha_scale, alpha_bias,
      mask_embedding, ve3, semantic_matrix, semantic_proj_w, catf)

    t = t3.reshape((N + 1) * BUCKETS, D)
    a2 = a.reshape(NW, RPW)
    g3 = g.reshape(NW, RPW // J, J)

    mesh = plsc.VectorSubcoreMesh(core_axis_name="c", subcore_axis_name="s")
    sc = functools.partial(
        pl.kernel, mesh=mesh,
        compiler_params=pltpu.CompilerParams(needs_layout_passes=False,
                                             use_tc_tiling_on_sc=False),
        out_type=jax.ShapeDtypeStruct((ROWS, D), jnp.float32),
        scratch_types=[
            pltpu.VMEM((RPW,), jnp.float32),
            pltpu.VMEM((RPW // J, J), jnp.int32),
            pltpu.VMEM((N, D), jnp.float32),
            pltpu.VMEM((CHUNK, D), jnp.float32),
            pltpu.SemaphoreType.DMA,
        ],
    )(_sc_body)
    out = sc(a2, g3, basis, t)
    return out.reshape(B, N, D)


# speedup vs baseline: 6.0227x; 1.0842x over previous
import functools

import jax
import jax.numpy as jnp
from jax import lax
from jax.experimental import pallas as pl
from jax.experimental.pallas import tpu as pltpu
from jax.experimental.pallas import tpu_sc as plsc

B = 4096
N = 100
D = 64
BUCKETS = 256
NW = 32
ROWS = B * N
RPW = ROWS // NW
CHUNK = 400
NCHUNK = RPW // CHUNK
J = 80
TPC = CHUNK // J


def _prep_body(alpha_ref, mask_ref, basis_ref, missing_ref, scale_ref,
               bias_ref, me_ref, ve3_ref, sm_ref, spw_ref, cat_ref,
               a_ref, g_ref, t3_ref):
    sem = lax.dot_general(sm_ref[...], spw_ref[...], (((1,), (1,)), ((), ())),
                          preferred_element_type=jnp.float32)
    me = me_ref[...]
    c1 = sem + me[1:2, :]
    c0 = sem + me[0:1, :] + missing_ref[...]
    catf = cat_ref[...]
    t3_ref[0:N] = (ve3_ref[...] * catf[:, None, None]
                   + c1[:, None, :])
    t3_ref[N:N + 1] = jnp.concatenate(
        [c0, jnp.zeros((BUCKETS - N, D), jnp.float32)], axis=0)[None]

    alpha = alpha_ref[...]
    mask = mask_ref[...]
    mask_f = mask.astype(jnp.float32)
    a_ref[...] = mask_f * (alpha * scale_ref[...][None, :]
                           + bias_ref[...][None, :])
    bucket = jnp.clip(
        jnp.round((jnp.clip(alpha, -1.0, 1.0) + 1.0) * 0.5 * (BUCKETS - 1)),
        0, BUCKETS - 1).astype(jnp.int32)
    n_iota = lax.broadcasted_iota(jnp.int32, alpha.shape, 1)
    g_ref[...] = jnp.where(mask == 1, n_iota * BUCKETS + bucket,
                           N * BUCKETS + n_iota)


NB = 3


def _sc_body(a_hbm, g_hbm, basis_hbm, t_hbm, out_hbm,
             a_v, g_v, basis_v, bufs, sem_g, sem_o):
    wid = lax.axis_index("s") * 2 + lax.axis_index("c")
    base = wid * RPW
    pltpu.sync_copy(a_hbm.at[wid], a_v)
    pltpu.sync_copy(g_hbm.at[wid], g_v)
    pltpu.sync_copy(basis_hbm, basis_v)

    def fire_gathers(c, b):
        for jj in range(TPC):
            pltpu.async_copy(
                t_hbm.at[g_v.at[c * TPC + jj]],
                bufs.at[b, pl.ds(jj * J, J)], sem_g)

    def wait_gathers(c, b):
        for jj in range(TPC):
            pltpu.make_async_copy(
                t_hbm.at[g_v.at[c * TPC + jj]],
                bufs.at[b, pl.ds(jj * J, J)], sem_g).wait()

    def out_desc(c, b):
        return pltpu.make_async_copy(
            bufs.at[b], out_hbm.at[pl.ds(base + c * CHUNK, CHUNK)], sem_o)

    fire_gathers(0, 0)

    def chunk_body(c, _):
        b = c % NB

        @pl.when(c >= 2)
        def _():
            out_desc(c - 2, (c - 2) % NB).wait()

        @pl.when(c + 1 < NCHUNK)
        def _():
            fire_gathers(c + 1, (c + 1) % NB)

        wait_gathers(c, b)

        def n_body(n, _):
            bvecs = [basis_v[n, pl.ds(k * 16, 16)] for k in range(4)]
            for j in range(CHUNK // N):
                i = n + j * N
                av = plsc.load_gather(
                    a_v, [jnp.full((16,), c * CHUNK + i, jnp.int32)])
                for k in range(4):
                    sl = pl.ds(k * 16, 16)
                    bufs[b, i, sl] = bufs[b, i, sl] + av * bvecs[k]
            return 0
        lax.fori_loop(0, N, n_body, 0)

        out_desc(c, b).start()
        return 0

    lax.fori_loop(0, NCHUNK, chunk_body, 0)
    out_desc(NCHUNK - 2, (NCHUNK - 2) % NB).wait()
    out_desc(NCHUNK - 1, (NCHUNK - 1) % NB).wait()


def kernel(alpha, mask, basis, missing_basis, alpha_scale, alpha_bias,
           mask_embedding, value_embedding, semantic_matrix, semantic_proj_w,
           categorical_value_mask):
    ve3 = value_embedding.reshape(N, BUCKETS, D)
    catf = categorical_value_mask.astype(jnp.float32)
    a, g, t3 = pl.pallas_call(
        _prep_body,
        out_shape=(
            jax.ShapeDtypeStruct((B, N), jnp.float32),
            jax.ShapeDtypeStruct((B, N), jnp.int32),
            jax.ShapeDtypeStruct((N + 1, BUCKETS, D), jnp.float32),
        ),
    )(alpha, mask, basis, missing_basis, alpha_scale, alpha_bias,
      mask_embedding, ve3, semantic_matrix, semantic_proj_w, catf)

    t = t3.reshape((N + 1) * BUCKETS, D)
    a2 = a.reshape(NW, RPW)
    g3 = g.reshape(NW, RPW // J, J)

    mesh = plsc.VectorSubcoreMesh(core_axis_name="c", subcore_axis_name="s")
    sc = functools.partial(
        pl.kernel, mesh=mesh,
        compiler_params=pltpu.CompilerParams(needs_layout_passes=False,
                                             use_tc_tiling_on_sc=False),
        out_type=jax.ShapeDtypeStruct((ROWS, D), jnp.float32),
        scratch_types=[
            pltpu.VMEM((RPW,), jnp.float32),
            pltpu.VMEM((RPW // J, J), jnp.int32),
            pltpu.VMEM((N, D), jnp.float32),
            pltpu.VMEM((NB, CHUNK, D), jnp.float32),
            pltpu.SemaphoreType.DMA,
            pltpu.SemaphoreType.DMA,
        ],
    )(_sc_body)
    out = sc(a2, g3, basis, t)
    return out.reshape(B, N, D)
